# baseline (speedup 1.0000x reference)
import functools

import jax
import jax.numpy as jnp
from jax import lax
from jax.experimental import pallas as pl
from jax.experimental.pallas import tpu as pltpu
from jax.experimental.pallas import tpu_sc as plsc

EPS_K = 0.1
L = 16
FILL_UNROLL = 10


def kernel(pred, target):
    n, c = pred.shape
    info = plsc.get_sparse_core_info()
    nc, ns = info.num_cores, info.num_subcores
    rows_per_core = n // nc
    blk_rows = 4
    rows_per_sub = n // (nc * ns)
    n_groups = rows_per_sub // 8
    smooth = EPS_K / (c - 1)
    hot = 1.0 - EPS_K

    mesh = plsc.VectorSubcoreMesh(core_axis_name="c", subcore_axis_name="s")

    @functools.partial(
        pl.kernel,
        out_type=jax.ShapeDtypeStruct((n, c), jnp.float32),
        mesh=mesh,
        scratch_types=[
            pltpu.VMEM((c,), jnp.float32),
            pltpu.VMEM_SHARED((blk_rows, c), jnp.float32),
            pltpu.VMEM((rows_per_sub,), jnp.int32),
            pltpu.VMEM((rows_per_sub, L), jnp.float32),
            pltpu.SemaphoreType.DMA,
            pltpu.SemaphoreType.DMA,
        ],
        compiler_params=pltpu.CompilerParams(use_tc_tiling_on_sc=False),
    )
    def sc_kernel(
        tgt_hbm, out_hbm, row_v, shared_v, tgt_v, win_v, sem_fill, sem_hot
    ):
        cid = lax.axis_index("c")
        sid = lax.axis_index("s")
        smoothv = jnp.full((L,), smooth, jnp.float32)

        def fill_body(i, carry):
            base = pl.multiple_of(i * (L * FILL_UNROLL), L * FILL_UNROLL)
            for j in range(FILL_UNROLL):
                row_v[pl.ds(base + j * L, L)] = smoothv
            return carry

        lax.fori_loop(0, c // (L * FILL_UNROLL), fill_body, 0)

        @pl.when(sid < blk_rows)
        def _():
            pltpu.sync_copy(row_v, shared_v.at[sid])

        plsc.subcore_barrier()

        fills = []
        for g in range(n_groups):
            r0 = cid * rows_per_core + 8 * sid + 8 * ns * g
            for q in range(8 // blk_rows):
                fills.append(
                    pltpu.async_copy(
                        shared_v,
                        out_hbm.at[pl.ds(r0 + q * blk_rows, blk_rows), :],
                        sem_fill,
                    )
                )

        iota = lax.iota(jnp.int32, L)
        for g in range(n_groups):
            r0 = cid * rows_per_core + 8 * sid + 8 * ns * g
            pltpu.sync_copy(
                tgt_hbm.at[pl.ds(r0, 8)],
                tgt_v.at[pl.ds(g * 8, 8)],
            )
        tscalars = []
        for h in range(rows_per_sub // L):
            tv = tgt_v[pl.ds(h * L, L)]
            for jj in range(L):
                tscalars.append(tv[jj])
        for p in range(rows_per_sub):
            win_v[p] = jnp.where(iota == tscalars[p] % L, hot, smooth)

        for d in fills:
            d.wait()

        hots = []
        for p in range(rows_per_sub):
            g, j2 = p // 8, p % 8
            r = cid * rows_per_core + 8 * sid + 8 * ns * g + j2
            b = pl.multiple_of((tscalars[p] // L) * L, L)
            hots.append(
                pltpu.async_copy(
                    win_v.at[p],
                    out_hbm.at[r, pl.ds(b, L)],
                    sem_hot,
                )
            )
        for d in hots:
            d.wait()

    return sc_kernel(target.astype(jnp.int32))


# speedup vs baseline: 2.3015x; 2.3015x over previous
import jax
import jax.numpy as jnp
from jax.experimental import pallas as pl
from jax.experimental.pallas import tpu as pltpu

EPS_K = 0.1
ROWS_CHUNK = 16
N_BUF = 8


def _fill_kernel(tgt_ref, out_ref, buf_ref, sem_ref):
    i = pl.program_id(0)
    n_chunks = pl.num_programs(0)
    c = out_ref.shape[1]
    smooth = jnp.float32(EPS_K / (c - 1))
    hot = jnp.float32(1.0 - EPS_K)
    slot = jax.lax.rem(i, N_BUF)
    buf = buf_ref.at[slot]

    @pl.when(i >= N_BUF)
    def _():
        prev = i - N_BUF
        pltpu.make_async_copy(
            buf,
            out_ref.at[pl.ds(prev * ROWS_CHUNK, ROWS_CHUNK), :],
            sem_ref.at[slot],
        ).wait()

    buf[...] = jnp.full((ROWS_CHUNK, c), smooth, jnp.float32)
    lane = jax.lax.broadcasted_iota(jnp.int32, (1, 128), 1)
    c_aligned = (c // 128) * 128
    tail = c - c_aligned
    for r in range(ROWS_CHUNK):
        t = tgt_ref[i * ROWS_CHUNK + r]
        base = (t // 128) * 128

        @pl.when(t < c_aligned)
        def _():
            buf[r : r + 1, pl.ds(base, 128)] = jnp.where(
                lane == t - base, hot, smooth
            )

        if tail:

            @pl.when(t >= c_aligned)
            def _():
                buf[r : r + 1, c_aligned:c] = jnp.where(
                    lane[:, :tail] == t - c_aligned, hot, smooth
                )

    pltpu.make_async_copy(
        buf,
        out_ref.at[pl.ds(i * ROWS_CHUNK, ROWS_CHUNK), :],
        sem_ref.at[slot],
    ).start()

    @pl.when(i == n_chunks - 1)
    def _():
        for k in range(N_BUF):
            chunk = n_chunks - N_BUF + k
            s = jax.lax.rem(chunk, N_BUF)
            pltpu.make_async_copy(
                buf_ref.at[s],
                out_ref.at[pl.ds(chunk * ROWS_CHUNK, ROWS_CHUNK), :],
                sem_ref.at[s],
            ).wait()


def kernel(pred, target):
    n, c = pred.shape
    n_chunks = n // ROWS_CHUNK
    return pl.pallas_call(
        _fill_kernel,
        grid_spec=pltpu.PrefetchScalarGridSpec(
            num_scalar_prefetch=1,
            grid=(n_chunks,),
            in_specs=[],
            out_specs=pl.BlockSpec(memory_space=pltpu.MemorySpace.HBM),
            scratch_shapes=[
                pltpu.VMEM((N_BUF, ROWS_CHUNK, c), jnp.float32),
                pltpu.SemaphoreType.DMA((N_BUF,)),
            ],
        ),
        out_shape=jax.ShapeDtypeStruct((n, c), jnp.float32),
        compiler_params=pltpu.CompilerParams(
            dimension_semantics=("arbitrary",),
        ),
    )(target.astype(jnp.int32))
